# baseline (speedup 1.0000x reference)
import functools

import jax
import jax.numpy as jnp
from jax import lax
from jax.experimental import pallas as pl
from jax.experimental.pallas import tpu as pltpu
from jax.experimental.pallas import tpu_sc as plsc

N_NODES = 10000
NPAD = 10240
D_FEAT = 10000
H_DIM = 64
C_DIM = 16
E_EDGES = 320000
EPAD = 327680
SUB = 128
NSUB = EPAD // SUB
JUNK = N_NODES
K_HOPS = 10

NC, NS, NW = 2, 16, 32
_MESH = plsc.VectorSubcoreMesh(core_axis_name="c", subcore_axis_name="s")


NPAD4 = 4 * NPAD
EPAD4 = 4 * EPAD
NSUB4 = EPAD4 // SUB
JUNK4 = 4 * N_NODES


@functools.partial(
    pl.kernel,
    out_type=(
        jax.ShapeDtypeStruct((NPAD4, C_DIM), jnp.float32),
        jax.ShapeDtypeStruct((NPAD, C_DIM), jnp.float32),
    ),
    mesh=_MESH,
    compiler_params=pltpu.CompilerParams(use_tc_tiling_on_sc=False),
    scratch_types=(
        pltpu.VMEM_SHARED((NPAD4, C_DIM), jnp.float32),
        pltpu.VMEM_SHARED((NPAD4, C_DIM), jnp.float32),
        pltpu.VMEM_SHARED((NPAD, C_DIM), jnp.float32),
        pltpu.VMEM((8, SUB), jnp.int32),
        pltpu.VMEM((8, SUB), jnp.int32),
        pltpu.VMEM((8, SUB), jnp.int32),
        pltpu.VMEM((8 * SUB, C_DIM), jnp.float32),
        pltpu.VMEM((8 * SUB, C_DIM), jnp.float32),
        pltpu.SemaphoreType.DMA,
        pltpu.SemaphoreType.DMA,
    ),
)
def _spmm_deg(rows2d, cols2d, w14_hbm, dst2d, zeros4, zeros16, ones_hbm,
              xp_out, degw_out,
              w1_sh, x_sh, deg_sh, rbuf, cbuf, dbuf, rows, onesb, gsem, ssem):
    cid = lax.axis_index("c")
    sid = lax.axis_index("s")

    @pl.when(cid == 0)
    def _body():
        rw = NPAD4 // NS
        r0 = sid * rw
        pltpu.sync_copy(w14_hbm.at[pl.ds(r0, rw)], w1_sh.at[pl.ds(r0, rw)])
        pltpu.sync_copy(zeros4.at[pl.ds(r0, rw)], x_sh.at[pl.ds(r0, rw)])
        rz = sid * (NPAD // NS)
        pltpu.sync_copy(zeros16.at[pl.ds(rz, NPAD // NS)],
                        deg_sh.at[pl.ds(rz, NPAD // NS)])
        pltpu.sync_copy(ones_hbm, onesb)
        plsc.subcore_barrier()

        def spmm_group(g, c):
            gs = sid * (NSUB4 // NS) + g * 8
            pltpu.sync_copy(rows2d.at[pl.ds(gs, 8)], rbuf)
            pltpu.sync_copy(cols2d.at[pl.ds(gs, 8)], cbuf)
            dls = [pltpu.async_copy(w1_sh.at[cbuf.at[b]],
                                    rows.at[pl.ds(b * SUB, SUB)], gsem)
                   for b in range(8)]
            for de in dls:
                de.wait()
            dls = [pltpu.async_copy(rows.at[pl.ds(b * SUB, SUB)],
                                    x_sh.at[rbuf.at[b]], ssem, add=True)
                   for b in range(8)]
            for de in dls:
                de.wait()
            return c
        lax.fori_loop(0, NSUB4 // NS // 8, spmm_group, 0)

        def deg_group(g, c):
            gs = sid * (NSUB // NS) + g * 8
            pltpu.sync_copy(dst2d.at[pl.ds(gs, 8)], dbuf)
            dls = [pltpu.async_copy(onesb.at[pl.ds(b * SUB, SUB)],
                                    deg_sh.at[dbuf.at[b]], ssem, add=True)
                   for b in range(8)]
            for de in dls:
                de.wait()
            return c
        lax.fori_loop(0, NSUB // NS // 8, deg_group, 0)
        plsc.subcore_barrier()

        pltpu.sync_copy(x_sh.at[pl.ds(r0, rw)], xp_out.at[pl.ds(r0, rw)])
        pltpu.sync_copy(deg_sh.at[pl.ds(rz, NPAD // NS)],
                        degw_out.at[pl.ds(rz, NPAD // NS)])


def _dense1_body(xp_ref, degw_ref, w2_ref, x0_ref, y0_ref, d16_ref, d9_ref,
                 d29_ref, h01_ref):
    xs = xp_ref[...]
    x = jnp.dot(jnp.maximum(xs, 0.0), w2_ref[...],
                preferred_element_type=jnp.float32)
    x0_ref[...] = x
    deg = degw_ref[:, 0:1] + 1.0
    dinv = lax.rsqrt(deg)
    db = jnp.broadcast_to(dinv, (NPAD, C_DIM))
    x0_ref[...] = x
    y0_ref[...] = x * db
    d16_ref[...] = db
    d9_ref[...] = 0.9 * db
    d29_ref[...] = jnp.broadcast_to(0.9 / deg, (NPAD, C_DIM))
    h01_ref[...] = 0.1 * x


def _dense1(xp, degw, W2):
    return pl.pallas_call(
        _dense1_body,
        out_shape=tuple(jax.ShapeDtypeStruct((NPAD, C_DIM), jnp.float32)
                        for _ in range(6)),
    )(xp, degw, W2)


@functools.partial(
    pl.kernel,
    out_type=jax.ShapeDtypeStruct((NPAD, C_DIM), jnp.float32),
    mesh=_MESH,
    compiler_params=pltpu.CompilerParams(use_tc_tiling_on_sc=False),
    scratch_types=(
        pltpu.VMEM_SHARED((NPAD, C_DIM), jnp.float32),
        pltpu.VMEM_SHARED((NPAD, C_DIM), jnp.float32),
        pltpu.VMEM((8, SUB), jnp.int32),
        pltpu.VMEM((8, SUB), jnp.int32),
        pltpu.VMEM((8 * SUB, C_DIM), jnp.float32),
        pltpu.VMEM((NPAD // NS, C_DIM), jnp.float32),
        pltpu.VMEM((NPAD // NS, C_DIM), jnp.float32),
        pltpu.VMEM((NPAD // NS, C_DIM), jnp.float32),
        pltpu.VMEM((NPAD // NS, C_DIM), jnp.float32),
        pltpu.VMEM((NPAD // NS, C_DIM), jnp.float32),
        pltpu.VMEM((NPAD // NS, C_DIM), jnp.float32),
        pltpu.VMEM((NPAD // NS, C_DIM), jnp.float32),
        pltpu.SemaphoreType.DMA,
        pltpu.SemaphoreType.DMA,
    ),
)
def _prop(src2d, dst2d, x0, y0, d16, d9, d29, h01, zeros16, xfin,
          y_sh, z_sh, sbuf, dbuf, rows, xbuf, h01b, d16b, d9b, d29b,
          zbuf, ybuf, gsem, ssem):
    cid = lax.axis_index("c")
    sid = lax.axis_index("s")
    rt = NPAD // NS
    r0 = sid * rt

    @pl.when(cid == 0)
    def _body():
        pltpu.sync_copy(x0.at[pl.ds(r0, rt)], xbuf)
        pltpu.sync_copy(h01.at[pl.ds(r0, rt)], h01b)
        pltpu.sync_copy(d16.at[pl.ds(r0, rt)], d16b)
        pltpu.sync_copy(d9.at[pl.ds(r0, rt)], d9b)
        pltpu.sync_copy(d29.at[pl.ds(r0, rt)], d29b)
        pltpu.sync_copy(y0.at[pl.ds(r0, rt)], y_sh.at[pl.ds(r0, rt)])
        pltpu.sync_copy(zeros16.at[pl.ds(r0, rt)], z_sh.at[pl.ds(r0, rt)])
        plsc.subcore_barrier()

        def hop(k, c):
            def grp(g, c2):
                p = lax.rem(g, 2) * 4
                gs = sid * 160 + g * 4
                pltpu.sync_copy(src2d.at[pl.ds(gs, 4)], sbuf.at[pl.ds(p, 4)])
                pltpu.sync_copy(dst2d.at[pl.ds(gs, 4)], dbuf.at[pl.ds(p, 4)])
                for b in range(4):
                    pltpu.async_copy(y_sh.at[sbuf.at[p + b]],
                                     rows.at[pl.ds((p + b) * SUB, SUB)], gsem)

                @pl.when(g > 0)
                def _drain_prev_scatter():
                    for b in range(4):
                        pltpu.make_async_copy(y0.at[pl.ds(0, SUB)],
                                              rows.at[pl.ds(0, SUB)],
                                              ssem).wait()
                for b in range(4):
                    pltpu.make_async_copy(y0.at[pl.ds(0, SUB)],
                                          rows.at[pl.ds(0, SUB)],
                                          gsem).wait()
                for b in range(4):
                    pltpu.async_copy(rows.at[pl.ds((p + b) * SUB, SUB)],
                                     z_sh.at[dbuf.at[p + b]], ssem, add=True)
                return c2
            lax.fori_loop(0, 40, grp, 0)
            for b in range(4):
                pltpu.make_async_copy(y0.at[pl.ds(0, SUB)],
                                      rows.at[pl.ds(0, SUB)], ssem).wait()
            plsc.subcore_barrier()

            pltpu.sync_copy(z_sh.at[pl.ds(r0, rt)], zbuf)
            pltpu.sync_copy(zeros16.at[pl.ds(r0, rt)], z_sh.at[pl.ds(r0, rt)])

            def upd(i, c2):
                xn = d9b[i] * zbuf[i] + d29b[i] * xbuf[i] + h01b[i]
                xbuf[i] = xn
                ybuf[i] = d16b[i] * xn
                return c2
            lax.fori_loop(0, rt, upd, 0)
            pltpu.sync_copy(ybuf, y_sh.at[pl.ds(r0, rt)])
            plsc.subcore_barrier()
            return c
        lax.fori_loop(0, K_HOPS, hop, 0)
        pltpu.sync_copy(xbuf, xfin.at[pl.ds(r0, rt)])


def _lsm_body(x_ref, o_ref):
    x = x_ref[...]
    m = jnp.max(x, axis=1, keepdims=True)
    ex = jnp.exp(x - m)
    lse = jnp.log(jnp.sum(ex, axis=1, keepdims=True)) + m
    o_ref[...] = x - lse


def _logsm(x):
    return pl.pallas_call(
        _lsm_body,
        out_shape=jax.ShapeDtypeStruct((NPAD, C_DIM), jnp.float32),
    )(x)


def _pad_idx(v, fill):
    pad = jnp.full((EPAD - E_EDGES,), fill, jnp.int32)
    return jnp.concatenate([v, pad]).reshape(NSUB, SUB)


def _pad_idx4(v, fill):
    pad = jnp.full((EPAD4 - 4 * E_EDGES,), fill, jnp.int32)
    return jnp.concatenate([v, pad]).reshape(NSUB4, SUB)


def kernel(attr_idx, edge_idx, n, d, W1, W2):
    del n, d
    four = jnp.arange(4, dtype=jnp.int32)
    ar4 = (attr_idx[0][:, None] * 4 + four).reshape(-1)
    ac4 = (attr_idx[1][:, None] * 4 + four).reshape(-1)
    ar2 = _pad_idx4(ar4, JUNK4)
    ac2 = _pad_idx4(ac4, 0)
    es2 = _pad_idx(edge_idx[0], 0)
    ed2 = _pad_idx(edge_idx[1], JUNK)
    w14 = jnp.concatenate(
        [W1, jnp.zeros((NPAD - D_FEAT, H_DIM), jnp.float32)]).reshape(NPAD4, C_DIM)
    zeros4 = jnp.zeros((NPAD4, C_DIM), jnp.float32)
    zeros16 = jnp.zeros((NPAD, C_DIM), jnp.float32)
    ones = jnp.ones((8 * SUB, C_DIM), jnp.float32)

    xp4, degw = _spmm_deg(ar2, ac2, w14, ed2, zeros4, zeros16, ones)
    xp = xp4.reshape(NPAD, H_DIM)
    x0, y0, d16, d9, d29, h01 = _dense1(xp, degw, W2)
    xfin = _prop(es2, ed2, x0, y0, d16, d9, d29, h01, zeros16)
    out = _logsm(xfin)
    return out[:N_NODES]


# speedup vs baseline: 1.1403x; 1.1403x over previous
import functools

import jax
import jax.numpy as jnp
from jax import lax
from jax.experimental import pallas as pl
from jax.experimental.pallas import tpu as pltpu
from jax.experimental.pallas import tpu_sc as plsc

N_NODES = 10000
NPAD = 10240
D_FEAT = 10000
H_DIM = 64
C_DIM = 16
E_EDGES = 320000
EPAD = 327680
SUB = 128
NSUB = EPAD // SUB
JUNK = N_NODES
K_HOPS = 10

NC, NS, NW = 2, 16, 32
_MESH = plsc.VectorSubcoreMesh(core_axis_name="c", subcore_axis_name="s")


NPAD4 = 4 * NPAD
EPAD4 = 4 * EPAD
NSUB4 = EPAD4 // SUB
JUNK4 = 4 * N_NODES


@functools.partial(
    pl.kernel,
    out_type=(
        jax.ShapeDtypeStruct((NPAD4, C_DIM), jnp.float32),
        jax.ShapeDtypeStruct((NPAD, C_DIM), jnp.float32),
    ),
    mesh=_MESH,
    compiler_params=pltpu.CompilerParams(use_tc_tiling_on_sc=False),
    scratch_types=(
        pltpu.VMEM_SHARED((NPAD4, C_DIM), jnp.float32),
        pltpu.VMEM_SHARED((NPAD4, C_DIM), jnp.float32),
        pltpu.VMEM_SHARED((NPAD, C_DIM), jnp.float32),
        pltpu.VMEM((8, SUB), jnp.int32),
        pltpu.VMEM((8, SUB), jnp.int32),
        pltpu.VMEM((8, SUB), jnp.int32),
        pltpu.VMEM((8 * SUB, C_DIM), jnp.float32),
        pltpu.VMEM((8 * SUB, C_DIM), jnp.float32),
        pltpu.SemaphoreType.DMA,
        pltpu.SemaphoreType.DMA,
    ),
)
def _spmm_deg(rows2d, cols2d, w14_hbm, dst2d, zeros4, zeros16, ones_hbm,
              xp_out, degw_out,
              w1_sh, x_sh, deg_sh, rbuf, cbuf, dbuf, rows, onesb, gsem, ssem):
    cid = lax.axis_index("c")
    sid = lax.axis_index("s")

    @pl.when(cid == 0)
    def _body():
        rw = NPAD4 // NS
        r0 = sid * rw
        pltpu.sync_copy(w14_hbm.at[pl.ds(r0, rw)], w1_sh.at[pl.ds(r0, rw)])
        pltpu.sync_copy(zeros4.at[pl.ds(r0, rw)], x_sh.at[pl.ds(r0, rw)])
        rz = sid * (NPAD // NS)
        pltpu.sync_copy(zeros16.at[pl.ds(rz, NPAD // NS)],
                        deg_sh.at[pl.ds(rz, NPAD // NS)])
        pltpu.sync_copy(ones_hbm, onesb)
        plsc.subcore_barrier()

        def spmm_group(g, c):
            gs = sid * (NSUB4 // NS) + g * 8
            pltpu.sync_copy(rows2d.at[pl.ds(gs, 8)], rbuf)
            pltpu.sync_copy(cols2d.at[pl.ds(gs, 8)], cbuf)
            dls = [pltpu.async_copy(w1_sh.at[cbuf.at[b]],
                                    rows.at[pl.ds(b * SUB, SUB)], gsem)
                   for b in range(8)]
            for de in dls:
                de.wait()
            dls = [pltpu.async_copy(rows.at[pl.ds(b * SUB, SUB)],
                                    x_sh.at[rbuf.at[b]], ssem, add=True)
                   for b in range(8)]
            for de in dls:
                de.wait()
            return c
        lax.fori_loop(0, NSUB4 // NS // 8, spmm_group, 0)

        def deg_group(g, c):
            gs = sid * (NSUB // NS) + g * 8
            pltpu.sync_copy(dst2d.at[pl.ds(gs, 8)], dbuf)
            dls = [pltpu.async_copy(onesb.at[pl.ds(b * SUB, SUB)],
                                    deg_sh.at[dbuf.at[b]], ssem, add=True)
                   for b in range(8)]
            for de in dls:
                de.wait()
            return c
        lax.fori_loop(0, NSUB // NS // 8, deg_group, 0)
        plsc.subcore_barrier()

        pltpu.sync_copy(x_sh.at[pl.ds(r0, rw)], xp_out.at[pl.ds(r0, rw)])
        pltpu.sync_copy(deg_sh.at[pl.ds(rz, NPAD // NS)],
                        degw_out.at[pl.ds(rz, NPAD // NS)])


def _dense1_body(xp_ref, degw_ref, w2_ref, x0_ref, y0_ref, d16_ref, d9_ref,
                 d29_ref, h01_ref):
    xs = xp_ref[...]
    x = jnp.dot(jnp.maximum(xs, 0.0), w2_ref[...],
                preferred_element_type=jnp.float32)
    x0_ref[...] = x
    deg = degw_ref[:, 0:1] + 1.0
    dinv = lax.rsqrt(deg)
    db = jnp.broadcast_to(dinv, (NPAD, C_DIM))
    x0_ref[...] = x
    y0_ref[...] = x * db
    d16_ref[...] = db
    d9_ref[...] = 0.9 * db
    d29_ref[...] = jnp.broadcast_to(0.9 / deg, (NPAD, C_DIM))
    h01_ref[...] = 0.1 * x


def _dense1(xp, degw, W2):
    return pl.pallas_call(
        _dense1_body,
        out_shape=tuple(jax.ShapeDtypeStruct((NPAD, C_DIM), jnp.float32)
                        for _ in range(6)),
    )(xp, degw, W2)


@functools.partial(
    pl.kernel,
    out_type=jax.ShapeDtypeStruct((NPAD, C_DIM), jnp.float32),
    mesh=_MESH,
    compiler_params=pltpu.CompilerParams(use_tc_tiling_on_sc=False),
    scratch_types=(
        pltpu.VMEM_SHARED((NPAD, C_DIM), jnp.float32),
        pltpu.VMEM_SHARED((NPAD, C_DIM), jnp.float32),
        pltpu.VMEM((16, SUB), jnp.int32),
        pltpu.VMEM((16, SUB), jnp.int32),
        pltpu.VMEM((16 * SUB, C_DIM), jnp.float32),
        pltpu.VMEM((NPAD // NS, C_DIM), jnp.float32),
        pltpu.VMEM((NPAD // NS, C_DIM), jnp.float32),
        pltpu.VMEM((NPAD // NS, C_DIM), jnp.float32),
        pltpu.VMEM((NPAD // NS, C_DIM), jnp.float32),
        pltpu.VMEM((NPAD // NS, C_DIM), jnp.float32),
        pltpu.VMEM((NPAD // NS, C_DIM), jnp.float32),
        pltpu.VMEM((NPAD // NS, C_DIM), jnp.float32),
        pltpu.SemaphoreType.DMA,
        pltpu.SemaphoreType.DMA,
    ),
)
def _prop(src2d, dst2d, x0, y0, d16, d9, d29, h01, zeros16, xfin,
          y_sh, z_sh, sbuf, dbuf, rows, xbuf, h01b, d16b, d9b, d29b,
          zbuf, ybuf, gsem, ssem):
    cid = lax.axis_index("c")
    sid = lax.axis_index("s")
    rt = NPAD // NS
    r0 = sid * rt

    @pl.when(cid == 0)
    def _body():
        pltpu.sync_copy(x0.at[pl.ds(r0, rt)], xbuf)
        pltpu.sync_copy(h01.at[pl.ds(r0, rt)], h01b)
        pltpu.sync_copy(d16.at[pl.ds(r0, rt)], d16b)
        pltpu.sync_copy(d9.at[pl.ds(r0, rt)], d9b)
        pltpu.sync_copy(d29.at[pl.ds(r0, rt)], d29b)
        pltpu.sync_copy(y0.at[pl.ds(r0, rt)], y_sh.at[pl.ds(r0, rt)])
        pltpu.sync_copy(zeros16.at[pl.ds(r0, rt)], z_sh.at[pl.ds(r0, rt)])
        plsc.subcore_barrier()

        def hop(k, c):
            def grp(g, c2):
                p = lax.rem(g, 2) * 8
                gs = sid * 160 + g * 8
                pltpu.sync_copy(src2d.at[pl.ds(gs, 8)], sbuf.at[pl.ds(p, 8)])
                pltpu.sync_copy(dst2d.at[pl.ds(gs, 8)], dbuf.at[pl.ds(p, 8)])
                for b in range(8):
                    pltpu.async_copy(y_sh.at[sbuf.at[p + b]],
                                     rows.at[pl.ds((p + b) * SUB, SUB)], gsem)

                @pl.when(g > 0)
                def _drain_prev_scatter():
                    for b in range(8):
                        pltpu.make_async_copy(y0.at[pl.ds(0, SUB)],
                                              rows.at[pl.ds(0, SUB)],
                                              ssem).wait()
                for b in range(8):
                    pltpu.make_async_copy(y0.at[pl.ds(0, SUB)],
                                          rows.at[pl.ds(0, SUB)],
                                          gsem).wait()
                for b in range(8):
                    pltpu.async_copy(rows.at[pl.ds((p + b) * SUB, SUB)],
                                     z_sh.at[dbuf.at[p + b]], ssem, add=True)
                return c2
            lax.fori_loop(0, 20, grp, 0)
            for b in range(8):
                pltpu.make_async_copy(y0.at[pl.ds(0, SUB)],
                                      rows.at[pl.ds(0, SUB)], ssem).wait()
            plsc.subcore_barrier()

            pltpu.sync_copy(z_sh.at[pl.ds(r0, rt)], zbuf)
            pltpu.sync_copy(zeros16.at[pl.ds(r0, rt)], z_sh.at[pl.ds(r0, rt)])

            def upd(i, c2):
                xn = d9b[i] * zbuf[i] + d29b[i] * xbuf[i] + h01b[i]
                xbuf[i] = xn
                ybuf[i] = d16b[i] * xn
                return c2
            lax.fori_loop(0, rt, upd, 0)
            pltpu.sync_copy(ybuf, y_sh.at[pl.ds(r0, rt)])
            plsc.subcore_barrier()
            return c
        lax.fori_loop(0, K_HOPS, hop, 0)
        pltpu.sync_copy(xbuf, xfin.at[pl.ds(r0, rt)])


def _lsm_body(x_ref, o_ref):
    x = x_ref[...]
    m = jnp.max(x, axis=1, keepdims=True)
    ex = jnp.exp(x - m)
    lse = jnp.log(jnp.sum(ex, axis=1, keepdims=True)) + m
    o_ref[...] = x - lse


def _logsm(x):
    return pl.pallas_call(
        _lsm_body,
        out_shape=jax.ShapeDtypeStruct((NPAD, C_DIM), jnp.float32),
    )(x)


def _pad_idx(v, fill):
    pad = jnp.full((EPAD - E_EDGES,), fill, jnp.int32)
    return jnp.concatenate([v, pad]).reshape(NSUB, SUB)


def _pad_idx4(v, fill):
    pad = jnp.full((EPAD4 - 4 * E_EDGES,), fill, jnp.int32)
    return jnp.concatenate([v, pad]).reshape(NSUB4, SUB)


def kernel(attr_idx, edge_idx, n, d, W1, W2):
    del n, d
    four = jnp.arange(4, dtype=jnp.int32)
    ar4 = (attr_idx[0][:, None] * 4 + four).reshape(-1)
    ac4 = (attr_idx[1][:, None] * 4 + four).reshape(-1)
    ar2 = _pad_idx4(ar4, JUNK4)
    ac2 = _pad_idx4(ac4, 0)
    es2 = _pad_idx(edge_idx[0], 0)
    ed2 = _pad_idx(edge_idx[1], JUNK)
    w14 = jnp.concatenate(
        [W1, jnp.zeros((NPAD - D_FEAT, H_DIM), jnp.float32)]).reshape(NPAD4, C_DIM)
    zeros4 = jnp.zeros((NPAD4, C_DIM), jnp.float32)
    zeros16 = jnp.zeros((NPAD, C_DIM), jnp.float32)
    ones = jnp.ones((8 * SUB, C_DIM), jnp.float32)

    xp4, degw = _spmm_deg(ar2, ac2, w14, ed2, zeros4, zeros16, ones)
    xp = xp4.reshape(NPAD, H_DIM)
    x0, y0, d16, d9, d29, h01 = _dense1(xp, degw, W2)
    xfin = _prop(es2, ed2, x0, y0, d16, d9, d29, h01, zeros16)
    out = _logsm(xfin)
    return out[:N_NODES]
